# baseline (speedup 1.0000x reference)
import functools

import jax
import jax.numpy as jnp
from jax import lax
from jax.experimental import pallas as pl
from jax.experimental.pallas import tpu as pltpu
from jax.experimental.pallas import tpu_sc as plsc

ROWS = 8192
SIZE = 1024
LANES = 16

_info = plsc.get_sparse_core_info()
NC = _info.num_cores
NS = _info.num_subcores
NW = NC * NS
ROWS_PER_W = ROWS // NW
BLK_ROWS = 32
NBLK = ROWS_PER_W // BLK_ROWS
CHUNKS = SIZE // LANES

_mesh = plsc.VectorSubcoreMesh(core_axis_name="c", subcore_axis_name="s")


@functools.partial(
    pl.kernel,
    mesh=_mesh,
    out_type=jax.ShapeDtypeStruct((ROWS * SIZE,), jnp.float32),
    scratch_types=[
        pltpu.VMEM((SIZE,), jnp.int32),
        pltpu.VMEM((BLK_ROWS * SIZE,), jnp.float32),
        pltpu.VMEM((BLK_ROWS * SIZE,), jnp.float32),
    ],
    compiler_params=pltpu.CompilerParams(needs_layout_passes=False),
)
def _permute_sc(z_hbm, p_hbm, out_hbm, p_v, zbuf, obuf):
    wid = lax.axis_index("s") * NC + lax.axis_index("c")
    base = wid * (ROWS_PER_W * SIZE)

    pltpu.sync_copy(p_hbm, p_v)

    def block_body(b, _):
        off = base + b * (BLK_ROWS * SIZE)
        pltpu.sync_copy(z_hbm.at[pl.ds(off, BLK_ROWS * SIZE)], zbuf)

        def chunk_body(j, _):
            idx0 = p_v[pl.ds(j * LANES, LANES)]

            def row_body(r, idx):
                val = plsc.load_gather(zbuf, [idx])
                obuf[pl.ds(r * SIZE + j * LANES, LANES)] = val
                return idx + SIZE

            lax.fori_loop(0, BLK_ROWS, row_body, idx0)
            return 0

        lax.fori_loop(0, CHUNKS, chunk_body, 0)
        pltpu.sync_copy(obuf, out_hbm.at[pl.ds(off, BLK_ROWS * SIZE)])
        return 0

    lax.fori_loop(0, NBLK, block_body, 0)


def kernel(z, p):
    zf = z.reshape(ROWS * SIZE)
    pi = p.astype(jnp.int32)
    out = _permute_sc(zf, pi)
    return out.reshape(ROWS, SIZE)
